# baseline (speedup 1.0000x reference)
import functools

import jax
import jax.numpy as jnp
import numpy as np
from jax import lax
from jax.experimental import pallas as pl
from jax.experimental.pallas import tpu as pltpu
from jax.experimental.pallas import tpu_sc as plsc

N = 10000
D = 128
K = 33

GN = 8
GE = GN * K
NG = N // GN
NC = 2
NS = 16
NW = NC * NS
TPW = (NG + NW - 1) // NW
ECH = 88

_PERM = np.empty(D, np.int32)
for _jj in range(D // 32):
    for _i in range(16):
        _PERM[32 * _jj + 2 * _i] = 32 * _jj + _i
        _PERM[32 * _jj + 2 * _i + 1] = 32 * _jj + 16 + _i
_PA = np.zeros((D, D // 2), np.float32)
_PB = np.zeros((D, D // 2), np.float32)
_PA[_PERM[0::2], np.arange(D // 2)] = 1.0
_PB[_PERM[1::2], np.arange(D // 2)] = 1.0


def _prep_body(h_ref, wt_ref, bt_ref, g_ref, b_ref, pa_ref, pb_ref,
               hb_ref, p_ref, q_ref):
    h = h_ref[...]
    mu = jnp.mean(h, axis=0, keepdims=True)
    var = jnp.mean((h - mu) ** 2, axis=0, keepdims=True)
    rstd = lax.rsqrt(var + 1e-5)
    hn = (h - mu) * (rstd * g_ref[...]) + b_ref[...]
    lo = lax.bitcast_convert_type(
        jnp.dot(hn, pa_ref[...],
                preferred_element_type=jnp.float32).astype(jnp.bfloat16),
        jnp.uint16).astype(jnp.int32)
    hi = lax.bitcast_convert_type(
        jnp.dot(hn, pb_ref[...],
                preferred_element_type=jnp.float32).astype(jnp.bfloat16),
        jnp.uint16).astype(jnp.int32)
    word = lo | (hi << 16)
    hb_ref[...] = jnp.concatenate([word, jnp.zeros_like(word)], axis=1)
    wt = wt_ref[...]
    m = lax.dot_general(wt, wt, (((1,), (1,)), ((), ())),
                        preferred_element_type=jnp.float32)
    p_ref[...] = jnp.dot(hn, m, preferred_element_type=jnp.float32)
    bt = bt_ref[...]
    wb = lax.dot_general(wt, bt, (((1,), (1,)), ((), ())),
                         preferred_element_type=jnp.float32)
    q = jnp.dot(hn, wb, preferred_element_type=jnp.float32)
    q_ref[...] = q + 0.5 * jnp.sum(bt * bt)


def _out_body(y_ref, wo_ref, bo_ref, o_ref):
    z = jnp.dot(y_ref[...], wo_ref[...],
                preferred_element_type=jnp.float32) + bo_ref[...]
    o_ref[...] = jnp.where(z >= 0, z, 0.01 * z)


def _sc_body(hn_hbm, p_hbm, q_hbm, col_hbm, a_hbm, y_hbm,
             idx_v, rows_v, qc_v, ps_v, qs_v, e_v, a_v, y_v, sem, osem, isem):
    wid = lax.axis_index("s") * NC + lax.axis_index("c")
    kio = lax.iota(jnp.int32, 16)
    nj = D // 16

    def copies(g, b):
        be = g * GE
        bn = g * GN
        cps = []
        for c in range(GE // ECH):
            sl = pl.ds(c * ECH, ECH)
            cps.append(pltpu.make_async_copy(
                hn_hbm.at[idx_v[b].at[sl]], rows_v[b].at[sl], sem[b]))
            cps.append(pltpu.make_async_copy(
                q_hbm.at[idx_v[b].at[sl]], qc_v[b].at[sl], sem[b]))
        cps.append(pltpu.make_async_copy(
            p_hbm.at[pl.ds(bn, GN)], ps_v[b], sem[b]))
        cps.append(pltpu.make_async_copy(
            q_hbm.at[pl.ds(bn, GN)], qs_v[b].at[pl.ds(0, GN)], sem[b]))
        return cps

    def out_copies(g, b):
        return [pltpu.make_async_copy(
                    a_v[b], a_hbm.at[pl.ds(g * GE, GE)], osem[b]),
                pltpu.make_async_copy(
                    y_v[b], y_hbm.at[pl.ds(g * GN, GN)], osem[b])]

    def idx_copy(g, b):
        return pltpu.make_async_copy(
            col_hbm.at[pl.ds(g * GE, GE)], idx_v[b], isem[b])

    def idx_start(g, b):
        @pl.when(g < NG)
        def _():
            idx_copy(g, b).start()

    def gather_start(g, b):
        @pl.when(g < NG)
        def _():
            idx_copy(g, b).wait()
            for cp in copies(g, b):
                cp.start()

    def wait_in(g, b):
        @pl.when(g < NG)
        def _():
            for cp in copies(g, b):
                cp.wait()

    def compute(g, b):
        @pl.when(g < NG)
        def _():
            @pl.when(g >= 2 * NW)
            def _():
                for cp in out_copies(g - 2 * NW, b):
                    cp.wait()

            rows, qc, av, yv = rows_v[b], qc_v[b], a_v[b], y_v[b]

            def node_body(n, _):
                row0 = n * K
                p = [ps_v[b][n, pl.ds(16 * j, 16)] for j in range(nj)]
                qi = plsc.load_gather(
                    qs_v[b], [jnp.full((16,), n, jnp.int32)])

                sv = jnp.zeros((16,), jnp.float32)
                evs = []
                for bb in range(3):
                    klen = 16 if bb < 2 else K - 32

                    def dot_k(kk, dv, _bb=bb):
                        e = row0 + _bb * 16 + kk
                        terms = []
                        for jj in range(nj // 2):
                            ra, rb = plsc.unpack(
                                plsc.bitcast(rows[e, pl.ds(16 * jj, 16)],
                                             jnp.bfloat16),
                                format=plsc.PackFormat.INTERLEAVED)
                            terms += [ra * p[2 * jj], rb * p[2 * jj + 1]]
                        while len(terms) > 1:
                            terms = [terms[i] + terms[i + 1]
                                     for i in range(0, len(terms) - 1, 2)
                                     ] + terms[len(terms) & ~1:]
                        d = jnp.full((16,), jnp.sum(terms[0]), jnp.float32)
                        return jnp.where(kio == kk, d, dv)

                    dv = lax.fori_loop(0, klen, dot_k,
                                       jnp.zeros((16,), jnp.float32),
                                       unroll=16)
                    ki = bb * 16 + kio
                    qcg = plsc.load_gather(
                        qc, [jnp.minimum(row0 + ki, GE - 1)])
                    ev = jnp.exp(1.0 / (1.0 + jnp.exp(-(dv + qcg + qi))))
                    ev = jnp.where(ki < K, ev, 0.0)
                    e_v[pl.ds(16 * bb, 16)] = ev
                    evs.append(ev)
                    sv = sv + ev
                rsv = 1.0 / jnp.full((16,), jnp.sum(sv), jnp.float32)
                for bb in range(3):
                    ki = bb * 16 + kio
                    plsc.store_scatter(av, [jnp.minimum(row0 + ki, GE - 1)],
                                       evs[bb] * rsv, mask=ki < K)

                def agg_k(k, acc):
                    e = row0 + k
                    eb = plsc.load_gather(e_v, [jnp.full((16,), k, jnp.int32)])
                    acc2 = []
                    for jj in range(nj // 2):
                        ra, rb = plsc.unpack(
                            plsc.bitcast(rows[e, pl.ds(16 * jj, 16)],
                                         jnp.bfloat16),
                            format=plsc.PackFormat.INTERLEAVED)
                        acc2 += [acc[2 * jj] + eb * ra,
                                 acc[2 * jj + 1] + eb * rb]
                    return tuple(acc2)

                acc = lax.fori_loop(
                    0, K, agg_k,
                    tuple(jnp.zeros((16,), jnp.float32) for _ in range(nj)),
                    unroll=8)
                for j in range(nj):
                    yv[n, pl.ds(16 * j, 16)] = acc[j] * rsv
                return 0

            lax.fori_loop(0, GN, node_body, 0)

            for cp in out_copies(g, b):
                cp.start()

    idx_start(wid, 0)
    gather_start(wid, 0)
    idx_start(NW + wid, 1)

    def trip2(t2, _):
        g0 = t2 * 2 * NW + wid
        g1 = g0 + NW
        wait_in(g0, 0)
        idx_start(g0 + 2 * NW, 0)
        gather_start(g1, 1)
        compute(g0, 0)
        wait_in(g1, 1)
        idx_start(g1 + 2 * NW, 1)
        gather_start(g0 + 2 * NW, 0)
        compute(g1, 1)
        return 0

    lax.fori_loop(0, TPW // 2, trip2, 0)

    tmax = (NG - 1 - wid) // NW
    for b in range(2):
        lb = tmax - ((tmax - b) % 2)

        @pl.when(lb >= 0)
        def _(b=b, lb=lb):
            for cp in out_copies(lb * NW + wid, b):
                cp.wait()


@jax.jit
def kernel(H, col, row, W_theta, b_theta, W_out, b_out, gamma, beta):
    del row
    hb32, p, q = pl.pallas_call(
        _prep_body,
        out_shape=[
            jax.ShapeDtypeStruct((N, D), jnp.int32),
            jax.ShapeDtypeStruct((N, D), jnp.float32),
            jax.ShapeDtypeStruct((N, 1), jnp.float32),
        ],
    )(H, W_theta, b_theta.reshape(1, D), gamma.reshape(1, D),
      beta.reshape(1, D), jnp.asarray(_PA), jnp.asarray(_PB))

    mesh = plsc.VectorSubcoreMesh(core_axis_name="c", subcore_axis_name="s",
                                  num_cores=NC, num_subcores=NS)
    sc = pl.kernel(
        _sc_body,
        out_type=[
            jax.ShapeDtypeStruct((N * K,), jnp.float32),
            jax.ShapeDtypeStruct((N, D), jnp.float32),
        ],
        mesh=mesh,
        compiler_params=pltpu.CompilerParams(needs_layout_passes=False),
        scratch_types=[
            [pltpu.VMEM((GE,), jnp.int32)] * 2,
            [pltpu.VMEM((GE, D), jnp.int32)] * 2,
            [pltpu.VMEM((GE,), jnp.float32)] * 2,
            [pltpu.VMEM((GN, D), jnp.float32)] * 2,
            [pltpu.VMEM((16,), jnp.float32)] * 2,
            pltpu.VMEM((48,), jnp.float32),
            [pltpu.VMEM((GE,), jnp.float32)] * 2,
            [pltpu.VMEM((GN, D), jnp.float32)] * 2,
            [pltpu.SemaphoreType.DMA] * 2,
            [pltpu.SemaphoreType.DMA] * 2,
            [pltpu.SemaphoreType.DMA] * 2,
        ],
    )
    a_flat, y = sc(hb32, p, q.reshape(N), col)

    out2 = pl.pallas_call(
        _out_body,
        out_shape=jax.ShapeDtypeStruct((N, D), jnp.float32),
    )(y, W_out, b_out.reshape(1, D))

    return out2, a_flat.reshape(N, K, 1)


# speedup vs baseline: 1.0986x; 1.0986x over previous
import functools

import jax
import jax.numpy as jnp
from jax import lax
from jax.experimental import pallas as pl
from jax.experimental.pallas import tpu as pltpu
from jax.experimental.pallas import tpu_sc as plsc

N = 10000
D = 128
K = 33

GN = 8
GE = GN * K
NG = N // GN
NC = 2
NS = 16
NW = NC * NS
TPW = (NG + NW - 1) // NW
ECH = 88


def _prep_body(h_ref, wt_ref, bt_ref, g_ref, b_ref, hn_ref, p_ref, q_ref):
    h = h_ref[...]
    mu = jnp.mean(h, axis=0, keepdims=True)
    var = jnp.mean((h - mu) ** 2, axis=0, keepdims=True)
    rstd = lax.rsqrt(var + 1e-5)
    hn = (h - mu) * (rstd * g_ref[...]) + b_ref[...]
    hn_ref[...] = hn
    wt = wt_ref[...]
    m = lax.dot_general(wt, wt, (((1,), (1,)), ((), ())),
                        preferred_element_type=jnp.float32)
    p_ref[...] = jnp.dot(hn, m, preferred_element_type=jnp.float32)
    bt = bt_ref[...]
    wb = lax.dot_general(wt, bt, (((1,), (1,)), ((), ())),
                         preferred_element_type=jnp.float32)
    q = jnp.dot(hn, wb, preferred_element_type=jnp.float32)
    q_ref[...] = q + 0.5 * jnp.sum(bt * bt)


def _out_body(y_ref, wo_ref, bo_ref, o_ref):
    z = jnp.dot(y_ref[...], wo_ref[...],
                preferred_element_type=jnp.float32) + bo_ref[...]
    o_ref[...] = jnp.where(z >= 0, z, 0.01 * z)


def _sc_body(hn_hbm, p_hbm, q_hbm, col_hbm, a_hbm, y_hbm,
             idx_v, rows_v, qc_v, ps_v, qs_v, e_v, a_v, y_v, sem, osem, isem):
    wid = lax.axis_index("s") * NC + lax.axis_index("c")
    kio = lax.iota(jnp.int32, 16)
    nj = D // 16

    def copies(g, b):
        be = g * GE
        bn = g * GN
        cps = []
        for c in range(GE // ECH):
            sl = pl.ds(c * ECH, ECH)
            cps.append(pltpu.make_async_copy(
                hn_hbm.at[idx_v[b].at[sl]], rows_v[b].at[sl], sem[b]))
            cps.append(pltpu.make_async_copy(
                q_hbm.at[idx_v[b].at[sl]], qc_v[b].at[sl], sem[b]))
        cps.append(pltpu.make_async_copy(
            p_hbm.at[pl.ds(bn, GN)], ps_v[b], sem[b]))
        cps.append(pltpu.make_async_copy(
            q_hbm.at[pl.ds(bn, GN)], qs_v[b].at[pl.ds(0, GN)], sem[b]))
        return cps

    def out_copies(g, b):
        return [pltpu.make_async_copy(
                    a_v[b], a_hbm.at[pl.ds(g * GE, GE)], osem[b]),
                pltpu.make_async_copy(
                    y_v[b], y_hbm.at[pl.ds(g * GN, GN)], osem[b])]

    def idx_copy(g, b):
        return pltpu.make_async_copy(
            col_hbm.at[pl.ds(g * GE, GE)], idx_v[b], isem[b])

    def idx_start(g, b):
        @pl.when(g < NG)
        def _():
            idx_copy(g, b).start()

    def gather_start(g, b):
        @pl.when(g < NG)
        def _():
            idx_copy(g, b).wait()
            for cp in copies(g, b):
                cp.start()

    def wait_in(g, b):
        @pl.when(g < NG)
        def _():
            for cp in copies(g, b):
                cp.wait()

    def compute(g, b):
        @pl.when(g < NG)
        def _():
            @pl.when(g >= 2 * NW)
            def _():
                for cp in out_copies(g - 2 * NW, b):
                    cp.wait()

            rows, qc, av, yv = rows_v[b], qc_v[b], a_v[b], y_v[b]

            def node_body(n, _):
                row0 = n * K
                p = [ps_v[b][n, pl.ds(16 * j, 16)] for j in range(nj)]
                qi = plsc.load_gather(
                    qs_v[b], [jnp.full((16,), n, jnp.int32)])

                sv = jnp.zeros((16,), jnp.float32)
                evs = []
                for bb in range(3):
                    klen = 16 if bb < 2 else K - 32

                    def dot_k(kk, dv, _bb=bb):
                        e = row0 + _bb * 16 + kk
                        terms = [rows[e, pl.ds(16 * j, 16)] * p[j]
                                 for j in range(nj)]
                        while len(terms) > 1:
                            terms = [terms[i] + terms[i + 1]
                                     for i in range(0, len(terms) - 1, 2)
                                     ] + terms[len(terms) & ~1:]
                        d = jnp.full((16,), jnp.sum(terms[0]), jnp.float32)
                        return jnp.where(kio == kk, d, dv)

                    dv = lax.fori_loop(0, klen, dot_k,
                                       jnp.zeros((16,), jnp.float32),
                                       unroll=16)
                    ki = bb * 16 + kio
                    qcg = plsc.load_gather(
                        qc, [jnp.minimum(row0 + ki, GE - 1)])
                    ev = jnp.exp(1.0 / (1.0 + jnp.exp(-(dv + qcg + qi))))
                    ev = jnp.where(ki < K, ev, 0.0)
                    e_v[pl.ds(16 * bb, 16)] = ev
                    evs.append(ev)
                    sv = sv + ev
                rsv = 1.0 / jnp.full((16,), jnp.sum(sv), jnp.float32)
                for bb in range(3):
                    ki = bb * 16 + kio
                    plsc.store_scatter(av, [jnp.minimum(row0 + ki, GE - 1)],
                                       evs[bb] * rsv, mask=ki < K)

                def agg_k(k, acc):
                    e = row0 + k
                    eb = plsc.load_gather(e_v, [jnp.full((16,), k, jnp.int32)])
                    return tuple(
                        acc[j] + eb * rows[e, pl.ds(16 * j, 16)]
                        for j in range(nj))

                acc = lax.fori_loop(
                    0, K, agg_k,
                    tuple(jnp.zeros((16,), jnp.float32) for _ in range(nj)),
                    unroll=8)
                for j in range(nj):
                    yv[n, pl.ds(16 * j, 16)] = acc[j] * rsv
                return 0

            lax.fori_loop(0, GN, node_body, 0)

            for cp in out_copies(g, b):
                cp.start()

    idx_start(wid, 0)
    gather_start(wid, 0)
    idx_start(NW + wid, 1)

    def trip2(t2, _):
        g0 = t2 * 2 * NW + wid
        g1 = g0 + NW
        wait_in(g0, 0)
        idx_start(g0 + 2 * NW, 0)
        gather_start(g1, 1)
        compute(g0, 0)
        wait_in(g1, 1)
        idx_start(g1 + 2 * NW, 1)
        gather_start(g0 + 2 * NW, 0)
        compute(g1, 1)
        return 0

    lax.fori_loop(0, TPW // 2, trip2, 0)

    tmax = (NG - 1 - wid) // NW
    for b in range(2):
        lb = tmax - ((tmax - b) % 2)

        @pl.when(lb >= 0)
        def _(b=b, lb=lb):
            for cp in out_copies(lb * NW + wid, b):
                cp.wait()


@jax.jit
def kernel(H, col, row, W_theta, b_theta, W_out, b_out, gamma, beta):
    del row
    hn, p, q = pl.pallas_call(
        _prep_body,
        out_shape=[
            jax.ShapeDtypeStruct((N, D), jnp.float32),
            jax.ShapeDtypeStruct((N, D), jnp.float32),
            jax.ShapeDtypeStruct((N, 1), jnp.float32),
        ],
    )(H, W_theta, b_theta.reshape(1, D), gamma.reshape(1, D),
      beta.reshape(1, D))

    mesh = plsc.VectorSubcoreMesh(core_axis_name="c", subcore_axis_name="s",
                                  num_cores=NC, num_subcores=NS)
    sc = pl.kernel(
        _sc_body,
        out_type=[
            jax.ShapeDtypeStruct((N * K,), jnp.float32),
            jax.ShapeDtypeStruct((N, D), jnp.float32),
        ],
        mesh=mesh,
        compiler_params=pltpu.CompilerParams(needs_layout_passes=False),
        scratch_types=[
            [pltpu.VMEM((GE,), jnp.int32)] * 2,
            [pltpu.VMEM((GE, D), jnp.float32)] * 2,
            [pltpu.VMEM((GE,), jnp.float32)] * 2,
            [pltpu.VMEM((GN, D), jnp.float32)] * 2,
            [pltpu.VMEM((16,), jnp.float32)] * 2,
            pltpu.VMEM((48,), jnp.float32),
            [pltpu.VMEM((GE,), jnp.float32)] * 2,
            [pltpu.VMEM((GN, D), jnp.float32)] * 2,
            [pltpu.SemaphoreType.DMA] * 2,
            [pltpu.SemaphoreType.DMA] * 2,
            [pltpu.SemaphoreType.DMA] * 2,
        ],
    )
    a_flat, y = sc(hn, p, q.reshape(N), col)

    out2 = pl.pallas_call(
        _out_body,
        out_shape=jax.ShapeDtypeStruct((N, D), jnp.float32),
    )(y, W_out, b_out.reshape(1, D))

    return out2, a_flat.reshape(N, K, 1)
